# baseline (speedup 1.0000x reference)
import functools

import jax
import jax.numpy as jnp
from jax import lax
from jax.experimental import pallas as pl
from jax.experimental.pallas import tpu as pltpu
from jax.experimental.pallas import tpu_sc as plsc


@functools.lru_cache(maxsize=None)
def _sc_gather_fn(B: int):
    info = plsc.get_sparse_core_info()
    nc, ns = 1, info.num_subcores
    nw = nc * ns
    per_w = B // nw
    lanes = info.num_lanes
    assert B % nw == 0 and per_w % lanes == 0

    mesh = plsc.VectorSubcoreMesh(
        core_axis_name="c", subcore_axis_name="s", num_cores=nc)

    @functools.partial(
        pl.kernel,
        out_type=jax.ShapeDtypeStruct((1, B), jnp.float32),
        mesh=mesh,
        compiler_params=pltpu.CompilerParams(
            needs_layout_passes=False, skip_device_barrier=True),
        scratch_types=[
            pltpu.VMEM((4,), jnp.float32),
            pltpu.VMEM((per_w,), jnp.int32),
            pltpu.VMEM((per_w,), jnp.float32),
            pltpu.SemaphoreType.DMA,
            pltpu.SemaphoreType.DMA,
        ],
    )
    def sc_gather(table_hbm, xs_hbm, g_hbm, tab_v, idx_v, g_v, sem_t, sem_x):
        wid = lax.axis_index("s") * nc + lax.axis_index("c")
        base = wid * per_w
        cp_t = pltpu.async_copy(table_hbm, tab_v, sem_t)
        cp_x = pltpu.async_copy(xs_hbm.at[0, pl.ds(base, per_w)], idx_v, sem_x)
        cp_t.wait()
        cp_x.wait()
        for i in range(per_w // lanes):
            sl = pl.ds(i * lanes, lanes)
            g_v[sl] = plsc.load_gather(tab_v, [idx_v[sl]])
        pltpu.sync_copy(g_v, g_hbm.at[0, pl.ds(base, per_w)])

    return sc_gather


def _tc_head_body(x_ref, t_ref, h_ref, w_ref, b_ref, o_ref):
    x = x_ref[...]
    g = jnp.where(x == 1, t_ref[1, 0], t_ref[0, 0])
    g = jnp.where(x == 2, t_ref[2, 0], g)
    g = jnp.where(x == 3, t_ref[3, 0], g)
    a = h_ref[0] * w_ref[0, 0] + b_ref[0]
    o_ref[0] = a + g


@functools.lru_cache(maxsize=None)
def _tc_head_fn(B: int, TI: int, R: int):
    return pl.pallas_call(
        _tc_head_body,
        grid=(R // TI,),
        in_specs=[
            pl.BlockSpec((1, B), lambda i: (0, 0)),
            pl.BlockSpec((4, 1), lambda i: (0, 0)),
            pl.BlockSpec((1, TI, 1), lambda i: (0, i, 0)),
            pl.BlockSpec((1, 1), lambda i: (0, 0)),
            pl.BlockSpec((1,), lambda i: (0,)),
        ],
        out_specs=pl.BlockSpec((1, TI, B), lambda i: (0, i, 0)),
        out_shape=jax.ShapeDtypeStruct((1, B, B), jnp.float32),
    )


def _tc_tail_body(prev_ref, g_ref, h_ref, w_ref, b_ref, o_ref):
    del prev_ref
    a = h_ref[0] * w_ref[0, 0] + b_ref[0]
    o_ref[0] = a + g_ref[0]


@functools.lru_cache(maxsize=None)
def _tc_tail_fn(B: int, TI: int, R: int):
    off = R // TI
    return pl.pallas_call(
        _tc_tail_body,
        grid=((B - R) // TI,),
        in_specs=[
            pl.BlockSpec(memory_space=pl.ANY),
            pl.BlockSpec((1, B), lambda i: (0, 0)),
            pl.BlockSpec((1, TI, 1), lambda i: (0, i + off, 0)),
            pl.BlockSpec((1, 1), lambda i: (0, 0)),
            pl.BlockSpec((1,), lambda i: (0,)),
        ],
        out_specs=pl.BlockSpec((1, TI, B), lambda i: (0, i + off, 0)),
        out_shape=jax.ShapeDtypeStruct((1, B, B), jnp.float32),
        input_output_aliases={0: 0},
    )


def kernel(xs, hs, emb_weight, lin_w, lin_b):
    B = xs.shape[1]
    TI, R = 512, 1024
    g = _sc_gather_fn(B)(emb_weight.reshape(-1), xs)
    head = _tc_head_fn(B, TI, R)(xs, emb_weight, hs, lin_w, lin_b)
    return _tc_tail_fn(B, TI, R)(head, g, hs, lin_w, lin_b)


# speedup vs baseline: 1.0784x; 1.0784x over previous
import functools

import jax
import jax.numpy as jnp
from jax import lax
from jax.experimental import pallas as pl
from jax.experimental.pallas import tpu as pltpu
from jax.experimental.pallas import tpu_sc as plsc


@functools.lru_cache(maxsize=None)
def _sc_gather_fn(B: int):
    info = plsc.get_sparse_core_info()
    nc, ns = 1, info.num_subcores
    nw = nc * ns
    per_w = B // nw
    lanes = info.num_lanes
    assert B % nw == 0 and per_w % lanes == 0

    mesh = plsc.VectorSubcoreMesh(
        core_axis_name="c", subcore_axis_name="s", num_cores=nc)

    @functools.partial(
        pl.kernel,
        out_type=jax.ShapeDtypeStruct((1, B), jnp.float32),
        mesh=mesh,
        compiler_params=pltpu.CompilerParams(
            needs_layout_passes=False, skip_device_barrier=True),
        scratch_types=[
            pltpu.VMEM((4,), jnp.float32),
            pltpu.VMEM((per_w,), jnp.int32),
            pltpu.VMEM((per_w,), jnp.float32),
            pltpu.SemaphoreType.DMA,
            pltpu.SemaphoreType.DMA,
        ],
    )
    def sc_gather(table_hbm, xs_hbm, g_hbm, tab_v, idx_v, g_v, sem_t, sem_x):
        wid = lax.axis_index("s") * nc + lax.axis_index("c")
        base = wid * per_w
        cp_t = pltpu.async_copy(table_hbm, tab_v, sem_t)
        cp_x = pltpu.async_copy(xs_hbm.at[0, pl.ds(base, per_w)], idx_v, sem_x)
        cp_t.wait()
        cp_x.wait()
        for i in range(per_w // lanes):
            sl = pl.ds(i * lanes, lanes)
            g_v[sl] = plsc.load_gather(tab_v, [idx_v[sl]])
        pltpu.sync_copy(g_v, g_hbm.at[0, pl.ds(base, per_w)])

    return sc_gather


def _row_term(h_ref, p_ref, i, TI):
    a_row = h_ref[0, pl.ds(i * TI, TI)] * p_ref[4] + p_ref[5]
    return jnp.reshape(a_row, (TI, 1))


def _tc_head_body(x_ref, h_ref, p_ref, o_ref, *, TI):
    i = pl.program_id(0)
    x = x_ref[...]
    g = jnp.where(x == 1, p_ref[1], p_ref[0])
    g = jnp.where(x == 2, p_ref[2], g)
    g = jnp.where(x == 3, p_ref[3], g)
    o_ref[0] = _row_term(h_ref, p_ref, i, TI) + g


@functools.lru_cache(maxsize=None)
def _tc_head_fn(B: int, TI: int, R: int):
    return pl.pallas_call(
        functools.partial(_tc_head_body, TI=TI),
        grid=(R // TI,),
        in_specs=[
            pl.BlockSpec((1, B), lambda i: (0, 0)),
            pl.BlockSpec((1, B), lambda i: (0, 0)),
            pl.BlockSpec(memory_space=pltpu.SMEM),
        ],
        out_specs=pl.BlockSpec((1, TI, B), lambda i: (0, i, 0)),
        out_shape=jax.ShapeDtypeStruct((1, B, B), jnp.float32),
    )


def _tc_tail_body(prev_ref, g_ref, h_ref, p_ref, o_ref, *, TI, off):
    del prev_ref
    i = pl.program_id(0) + off
    o_ref[0] = _row_term(h_ref, p_ref, i, TI) + g_ref[...]


@functools.lru_cache(maxsize=None)
def _tc_tail_fn(B: int, TI: int, R: int):
    off = R // TI
    return pl.pallas_call(
        functools.partial(_tc_tail_body, TI=TI, off=off),
        grid=((B - R) // TI,),
        in_specs=[
            pl.BlockSpec(memory_space=pl.ANY),
            pl.BlockSpec((1, B), lambda i: (0, 0)),
            pl.BlockSpec((1, B), lambda i: (0, 0)),
            pl.BlockSpec(memory_space=pltpu.SMEM),
        ],
        out_specs=pl.BlockSpec((1, TI, B), lambda i: (0, i + off, 0)),
        out_shape=jax.ShapeDtypeStruct((1, B, B), jnp.float32),
        input_output_aliases={0: 0},
    )


def kernel(xs, hs, emb_weight, lin_w, lin_b):
    B = xs.shape[1]
    TI, R = 512, 1024
    hs_row = hs.reshape(1, B)
    params = jnp.concatenate(
        [emb_weight.reshape(4), lin_w.reshape(1), lin_b.reshape(1)])
    g = _sc_gather_fn(B)(emb_weight.reshape(-1), xs)
    head = _tc_head_fn(B, TI, R)(xs, hs_row, params)
    return _tc_tail_fn(B, TI, R)(head, g, hs_row, params)


# speedup vs baseline: 1.0815x; 1.0029x over previous
import functools

import jax
import jax.numpy as jnp
from jax import lax
from jax.experimental import pallas as pl
from jax.experimental.pallas import tpu as pltpu
from jax.experimental.pallas import tpu_sc as plsc


@functools.lru_cache(maxsize=None)
def _sc_gather_fn(B: int):
    info = plsc.get_sparse_core_info()
    nc, ns = 1, info.num_subcores
    nw = nc * ns
    per_w = B // nw
    lanes = info.num_lanes
    assert B % nw == 0 and per_w % lanes == 0

    mesh = plsc.VectorSubcoreMesh(
        core_axis_name="c", subcore_axis_name="s", num_cores=nc)

    @functools.partial(
        pl.kernel,
        out_type=jax.ShapeDtypeStruct((1, B), jnp.float32),
        mesh=mesh,
        compiler_params=pltpu.CompilerParams(
            needs_layout_passes=False, skip_device_barrier=True),
        scratch_types=[
            pltpu.VMEM((4,), jnp.float32),
            pltpu.VMEM((per_w,), jnp.int32),
            pltpu.VMEM((per_w,), jnp.float32),
            pltpu.SemaphoreType.DMA,
            pltpu.SemaphoreType.DMA,
        ],
    )
    def sc_gather(table_hbm, xs_hbm, g_hbm, tab_v, idx_v, g_v, sem_t, sem_x):
        wid = lax.axis_index("s") * nc + lax.axis_index("c")
        base = wid * per_w
        cp_t = pltpu.async_copy(table_hbm, tab_v, sem_t)
        cp_x = pltpu.async_copy(xs_hbm.at[0, pl.ds(base, per_w)], idx_v, sem_x)
        cp_t.wait()
        cp_x.wait()
        for i in range(per_w // lanes):
            sl = pl.ds(i * lanes, lanes)
            g_v[sl] = plsc.load_gather(tab_v, [idx_v[sl]])
        pltpu.sync_copy(g_v, g_hbm.at[0, pl.ds(base, per_w)])

    return sc_gather


def _row_term(h_ref, w_ref, b_ref, i, TI):
    a_row = h_ref[0, pl.ds(i * TI, TI)] * w_ref[0, 0] + b_ref[0]
    return jnp.reshape(a_row, (TI, 1))


def _tc_head_body(x_ref, h_ref, t_ref, w_ref, b_ref, o_ref, *, TI):
    i = pl.program_id(0)
    x = x_ref[...]
    g = jnp.where(x == 1, t_ref[1, 0], t_ref[0, 0])
    g = jnp.where(x == 2, t_ref[2, 0], g)
    g = jnp.where(x == 3, t_ref[3, 0], g)
    o_ref[0] = _row_term(h_ref, w_ref, b_ref, i, TI) + g


@functools.lru_cache(maxsize=None)
def _tc_head_fn(B: int, TI: int, R: int):
    return pl.pallas_call(
        functools.partial(_tc_head_body, TI=TI),
        grid=(R // TI,),
        in_specs=[
            pl.BlockSpec((1, B), lambda i: (0, 0)),
            pl.BlockSpec((1, B), lambda i: (0, 0)),
            pl.BlockSpec(memory_space=pltpu.SMEM),
            pl.BlockSpec(memory_space=pltpu.SMEM),
            pl.BlockSpec(memory_space=pltpu.SMEM),
        ],
        out_specs=pl.BlockSpec((1, TI, B), lambda i: (0, i, 0)),
        out_shape=jax.ShapeDtypeStruct((1, B, B), jnp.float32),
    )


def _tc_tail_body(prev_ref, g_ref, h_ref, w_ref, b_ref, o_ref, *, TI, off):
    del prev_ref
    i = pl.program_id(0) + off
    o_ref[0] = _row_term(h_ref, w_ref, b_ref, i, TI) + g_ref[...]


@functools.lru_cache(maxsize=None)
def _tc_tail_fn(B: int, TI: int, R: int):
    off = R // TI
    return pl.pallas_call(
        functools.partial(_tc_tail_body, TI=TI, off=off),
        grid=((B - R) // TI,),
        in_specs=[
            pl.BlockSpec(memory_space=pl.ANY),
            pl.BlockSpec((1, B), lambda i: (0, 0)),
            pl.BlockSpec((1, B), lambda i: (0, 0)),
            pl.BlockSpec(memory_space=pltpu.SMEM),
            pl.BlockSpec(memory_space=pltpu.SMEM),
        ],
        out_specs=pl.BlockSpec((1, TI, B), lambda i: (0, i + off, 0)),
        out_shape=jax.ShapeDtypeStruct((1, B, B), jnp.float32),
        input_output_aliases={0: 0},
    )


def kernel(xs, hs, emb_weight, lin_w, lin_b):
    B = xs.shape[1]
    TI, R = 512, 512
    hs_row = hs.reshape(1, B)
    g = _sc_gather_fn(B)(emb_weight.reshape(-1), xs)
    head = _tc_head_fn(B, TI, R)(xs, hs_row, emb_weight, lin_w, lin_b)
    return _tc_tail_fn(B, TI, R)(head, g, hs_row, lin_w, lin_b)
